# baseline (speedup 1.0000x reference)
import functools
import math

import jax
import jax.numpy as jnp
from jax import lax
from jax.experimental import pallas as pl
from jax.experimental.pallas import tpu as pltpu
from jax.experimental.pallas import tpu_sc as plsc

VOCAB = 1000000
D_MODEL = 64
SCALE = math.sqrt(D_MODEL)

NC = 2
NS = 16
NW = NC * NS

N_TCOL = VOCAB // 128
TAIL_I0 = N_TCOL * 128
CONV_ROWS = VOCAB // 2

B_DIM = 4096
J_DIM = 200
N_BBLK = B_DIM // 128


def _conv_body(tbl_t, tail2, conv, in_v, out_v, sems, osems):
    wid = lax.axis_index("s") * NC + lax.axis_index("c")
    iota = lax.iota(jnp.int32, 16)
    prow = [(iota + 16 * g) >> 1 for g in range(16)]
    pcol = [((iota + 16 * g) & 1) << 6 for g in range(16)]
    colv = [iota + 16 * g for g in range(16)]

    def transpose_col(b):
        in_ref, out_ref = in_v[b], out_v[b]

        @plsc.parallel_loop(0, 16, unroll=2)
        def _(c):
            dbase = (iota + c) & 15
            for g in range(16):
                for m in range(4):
                    dvec = dbase + 16 * m
                    v = plsc.load_gather(in_ref, [dvec, colv[g]]) * SCALE
                    plsc.store_scatter(out_ref, [prow[g], pcol[g] + dvec], v)

    def fire(k, b):
        dc = wid + NW * k
        pltpu.async_copy(tbl_t.at[:, pl.ds(dc * 256, 256)], in_v[b], sems[b])

    def drain(b):
        pltpu.make_async_copy(tbl_t.at[:, pl.ds(0, 256)], in_v[b],
                              sems[b]).wait()

    def drain_store(b):
        pltpu.make_async_copy(conv.at[pl.ds(0, 128)], out_v[b],
                              osems[b]).wait()

    n_dcol = N_TCOL // 2
    n_k = (n_dcol - 1) // NW + 1

    fire(0, 0)

    def pair_body(cp, _):
        for b in range(2):
            k = 2 * cp + b
            dc = wid + NW * k

            @pl.when(dc < n_dcol)
            def _():
                @pl.when(k >= 2)
                def _():
                    drain_store(b)

                @pl.when(dc + NW < n_dcol)
                def _():
                    fire(k + 1, 1 - b)

                drain(b)
                transpose_col(b)
                pltpu.async_copy(out_v[b], conv.at[pl.ds(dc * 128, 128)],
                                 osems[b])
        return ()

    lax.fori_loop(0, (n_k + 1) // 2, pair_body, ())
    drain_store(0)
    drain_store(1)

    @pl.when(wid == 0)
    def _():
        pltpu.sync_copy(tail2, out_v[0].at[pl.ds(0, 32)])

        @plsc.parallel_loop(0, 32, unroll=8)
        def _(p):
            for g in range(8):
                sl = pl.ds(16 * g, 16)
                out_v[0][p, sl] = out_v[0][p, sl] * SCALE

        pltpu.sync_copy(out_v[0].at[pl.ds(0, 32)],
                        conv.at[pl.ds(TAIL_I0 // 2, 32)])


def _gather_body(xt, conv2, o5, idx_v, rows_v, out_v, isems, gsems, osems):
    wid = lax.axis_index("s") * NC + lax.axis_index("c")
    iota = lax.iota(jnp.int32, 16)
    tvecs = [iota + 16 * g for g in range(8)]

    def fire_idx(j, b):
        pltpu.async_copy(xt.at[pl.ds(j, 1), pl.ds(wid * 128, 128)],
                         idx_v[b], isems[b])

    def drain_idx(b):
        pltpu.make_async_copy(xt.at[pl.ds(0, 1), pl.ds(0, 128)], idx_v[b],
                              isems[b]).wait()

    def fire_gather(b):
        pltpu.async_copy(conv2.at[idx_v[b].at[0]], rows_v[b], gsems[b])

    def drain_gather(b):
        pltpu.make_async_copy(conv2.at[pl.ds(0, 128)], rows_v[b],
                              gsems[b]).wait()

    def drain_store(b):
        for tr in range(8):
            pltpu.make_async_copy(o5.at[0, tr, 0], out_v[b].at[pl.ds(8 * tr, 8)],
                                  osems[b]).wait()

    def extract(b):
        rv, ov = rows_v[b], out_v[b]

        @plsc.parallel_loop(0, 16, unroll=2)
        def _(c):
            dbase = (iota + c) & 15
            for g in range(8):
                tv = tvecs[g]
                for m in range(4):
                    dvec = dbase + 16 * m
                    v = plsc.load_gather(rv, [tv, dvec])
                    plsc.store_scatter(ov, [dvec, tv], v)

    def store(j, b):
        for tr in range(8):
            pltpu.async_copy(out_v[b].at[pl.ds(8 * tr, 8)],
                             o5.at[j, tr, wid], osems[b])

    fire_idx(0, 0)
    drain_idx(0)
    fire_gather(0)
    fire_idx(1, 1)

    def pair_body(cp, _):
        for b in range(2):
            j = 2 * cp + b

            @pl.when(j + 1 < J_DIM)
            def _():
                drain_idx(1 - b)
                fire_gather(1 - b)

            @pl.when(j + 2 < J_DIM)
            def _():
                fire_idx(j + 2, b)

            @pl.when(j >= 2)
            def _():
                drain_store(b)

            drain_gather(b)
            extract(b)
            store(j, b)
        return ()

    lax.fori_loop(0, J_DIM // 2, pair_body, ())
    drain_store(0)
    drain_store(1)


def kernel(x, table):
    mesh = plsc.VectorSubcoreMesh(core_axis_name="c", subcore_axis_name="s")

    conv = pl.kernel(
        _conv_body,
        out_type=jax.ShapeDtypeStruct((CONV_ROWS, 128), jnp.float32),
        mesh=mesh,
        scratch_types=[
            [pltpu.VMEM((D_MODEL, 256), jnp.float32) for _ in range(2)],
            [pltpu.VMEM((128, 128), jnp.float32) for _ in range(2)],
            [pltpu.SemaphoreType.DMA for _ in range(2)],
            [pltpu.SemaphoreType.DMA for _ in range(2)],
        ],
        compiler_params=pltpu.CompilerParams(use_tc_tiling_on_sc=True,
                                             needs_layout_passes=False),
    )(table.T, table[TAIL_I0:].reshape(32, 128))

    conv2 = conv.reshape(VOCAB, D_MODEL)
    xt = x.T.astype(jnp.int32)

    o5 = pl.kernel(
        _gather_body,
        out_type=jax.ShapeDtypeStruct((J_DIM, 8, N_BBLK, 8, 128), jnp.float32),
        mesh=mesh,
        scratch_types=[
            [pltpu.VMEM((1, 128), jnp.int32) for _ in range(2)],
            [pltpu.VMEM((128, D_MODEL), jnp.float32) for _ in range(2)],
            [pltpu.VMEM((D_MODEL, 128), jnp.float32) for _ in range(2)],
            [pltpu.SemaphoreType.DMA for _ in range(2)],
            [pltpu.SemaphoreType.DMA for _ in range(2)],
            [pltpu.SemaphoreType.DMA for _ in range(2)],
        ],
        compiler_params=pltpu.CompilerParams(use_tc_tiling_on_sc=False,
                                             needs_layout_passes=False),
    )(xt, conv2)

    return o5.transpose(2, 4, 0, 1, 3).reshape(B_DIM, J_DIM, D_MODEL)


# speedup vs baseline: 1.0965x; 1.0965x over previous
import functools
import math

import jax
import jax.numpy as jnp
from jax import lax
from jax.experimental import pallas as pl
from jax.experimental.pallas import tpu as pltpu
from jax.experimental.pallas import tpu_sc as plsc

VOCAB = 1000000
D_MODEL = 64
SCALE = math.sqrt(D_MODEL)

NC = 2
NS = 16
NW = NC * NS

N_TCOL = VOCAB // 128
TAIL_I0 = N_TCOL * 128
CONV_ROWS = VOCAB // 2

B_DIM = 4096
J_DIM = 200
N_BBLK = B_DIM // 128


def _conv_body(tbl_t, tail2, conv, in_v, out_v, sems, osems):
    wid = lax.axis_index("s") * NC + lax.axis_index("c")
    iota = lax.iota(jnp.int32, 16)
    prow = [(iota + 16 * g) >> 1 for g in range(8)]
    pcol = [((iota + 16 * g) & 1) << 6 for g in range(8)]
    colv = [iota + 16 * g for g in range(8)]

    def transpose_col(b):
        in_ref, out_ref = in_v[b], out_v[b]

        @plsc.parallel_loop(0, 16, unroll=2)
        def _(c):
            dbase = (iota + c) & 15
            for g in range(8):
                for m in range(4):
                    dvec = dbase + 16 * m
                    v = plsc.load_gather(in_ref, [dvec, colv[g]]) * SCALE
                    plsc.store_scatter(out_ref, [prow[g], pcol[g] + dvec], v)

    def fire(k, b):
        tc = wid + NW * k
        for tr in range(8):
            pltpu.async_copy(tbl_t.at[pl.ds(8 * tr, 8), pl.ds(tc * 128, 128)],
                             in_v[b].at[pl.ds(8 * tr, 8)], sems[b])

    def drain(b):
        pltpu.make_async_copy(tbl_t.at[:, pl.ds(0, 128)], in_v[b],
                              sems[b]).wait()

    def drain_store(b):
        pltpu.make_async_copy(tbl_t.at[:, pl.ds(0, 128)], out_v[b],
                              osems[b]).wait()

    n_k = (N_TCOL - 1) // NW + 1

    fire(0, 0)

    def pair_body(cp, _):
        for b in range(2):
            k = 2 * cp + b
            tc = wid + NW * k

            @pl.when(tc < N_TCOL)
            def _():
                @pl.when(k >= 2)
                def _():
                    drain_store(b)

                @pl.when(tc + NW < N_TCOL)
                def _():
                    fire(k + 1, 1 - b)

                drain(b)
                transpose_col(b)
                pltpu.async_copy(out_v[b], conv.at[pl.ds(tc * 64, 64)],
                                 osems[b])
        return ()

    lax.fori_loop(0, (n_k + 1) // 2, pair_body, ())
    drain_store(0)
    drain_store(1)

    @pl.when(wid == 0)
    def _():
        pltpu.sync_copy(tail2, in_v[0].at[pl.ds(0, 32)])

        @plsc.parallel_loop(0, 32, unroll=8)
        def _(p):
            for g in range(8):
                sl = pl.ds(16 * g, 16)
                in_v[0][p, sl] = in_v[0][p, sl] * SCALE

        pltpu.sync_copy(in_v[0].at[pl.ds(0, 32)],
                        conv.at[pl.ds(TAIL_I0 // 2, 32)])


def _gather_body(xt, conv2, o5, idx_v, rows_v, out_v, isems, gsems, osems):
    wid = lax.axis_index("s") * NC + lax.axis_index("c")
    iota = lax.iota(jnp.int32, 16)
    tvecs = [iota + 16 * g for g in range(8)]

    def fire_idx(j, b):
        pltpu.async_copy(xt.at[pl.ds(j, 1), pl.ds(wid * 128, 128)],
                         idx_v[b], isems[b])

    def drain_idx(b):
        pltpu.make_async_copy(xt.at[pl.ds(0, 1), pl.ds(0, 128)], idx_v[b],
                              isems[b]).wait()

    def fire_gather(b):
        pltpu.async_copy(conv2.at[idx_v[b].at[0]], rows_v[b], gsems[b])

    def drain_gather(b):
        pltpu.make_async_copy(conv2.at[pl.ds(0, 128)], rows_v[b],
                              gsems[b]).wait()

    def drain_store(b):
        for tr in range(8):
            pltpu.make_async_copy(o5.at[0, tr, 0], out_v[b].at[pl.ds(8 * tr, 8)],
                                  osems[b]).wait()

    def extract(b):
        rv, ov = rows_v[b], out_v[b]

        @plsc.parallel_loop(0, 16, unroll=2)
        def _(c):
            dbase = (iota + c) & 15
            for g in range(8):
                tv = tvecs[g]
                for m in range(4):
                    dvec = dbase + 16 * m
                    v = plsc.load_gather(rv, [tv, dvec])
                    plsc.store_scatter(ov, [dvec, tv], v)

    def store(j, b):
        for tr in range(8):
            pltpu.async_copy(out_v[b].at[pl.ds(8 * tr, 8)],
                             o5.at[j, tr, wid], osems[b])

    fire_idx(0, 0)
    drain_idx(0)
    fire_gather(0)
    fire_idx(1, 1)

    def pair_body(cp, _):
        for b in range(2):
            j = 2 * cp + b

            @pl.when(j + 1 < J_DIM)
            def _():
                drain_idx(1 - b)
                fire_gather(1 - b)

            @pl.when(j + 2 < J_DIM)
            def _():
                fire_idx(j + 2, b)

            @pl.when(j >= 2)
            def _():
                drain_store(b)

            drain_gather(b)
            extract(b)
            store(j, b)
        return ()

    lax.fori_loop(0, J_DIM // 2, pair_body, ())
    drain_store(0)
    drain_store(1)


def kernel(x, table):
    mesh = plsc.VectorSubcoreMesh(core_axis_name="c", subcore_axis_name="s")

    conv = pl.kernel(
        _conv_body,
        out_type=jax.ShapeDtypeStruct((CONV_ROWS, 128), jnp.float32),
        mesh=mesh,
        scratch_types=[
            [pltpu.VMEM((D_MODEL, 128), jnp.float32) for _ in range(2)],
            [pltpu.VMEM((D_MODEL, 128), jnp.float32) for _ in range(2)],
            [pltpu.SemaphoreType.DMA for _ in range(2)],
            [pltpu.SemaphoreType.DMA for _ in range(2)],
        ],
        compiler_params=pltpu.CompilerParams(use_tc_tiling_on_sc=True,
                                             needs_layout_passes=False),
    )(table.T, table[TAIL_I0:].reshape(32, 128))

    conv2 = conv.reshape(VOCAB, D_MODEL)
    xt = x.T.astype(jnp.int32)

    o5 = pl.kernel(
        _gather_body,
        out_type=jax.ShapeDtypeStruct((J_DIM, 8, N_BBLK, 8, 128), jnp.float32),
        mesh=mesh,
        scratch_types=[
            [pltpu.VMEM((1, 128), jnp.int32) for _ in range(2)],
            [pltpu.VMEM((128, D_MODEL), jnp.float32) for _ in range(2)],
            [pltpu.VMEM((D_MODEL, 128), jnp.float32) for _ in range(2)],
            [pltpu.SemaphoreType.DMA for _ in range(2)],
            [pltpu.SemaphoreType.DMA for _ in range(2)],
            [pltpu.SemaphoreType.DMA for _ in range(2)],
        ],
        compiler_params=pltpu.CompilerParams(use_tc_tiling_on_sc=False,
                                             needs_layout_passes=False),
    )(xt, conv2)

    return o5.transpose(2, 4, 0, 1, 3).reshape(B_DIM, J_DIM, D_MODEL)


# speedup vs baseline: 1.1144x; 1.0163x over previous
import functools
import math

import jax
import jax.numpy as jnp
from jax import lax
from jax.experimental import pallas as pl
from jax.experimental.pallas import tpu as pltpu
from jax.experimental.pallas import tpu_sc as plsc

VOCAB = 1000000
D_MODEL = 64
SCALE = math.sqrt(D_MODEL)

NC = 2
NS = 16
NW = NC * NS

N_TCOL = VOCAB // 128
TAIL_I0 = N_TCOL * 128
CONV_ROWS = VOCAB // 2

B_DIM = 4096
J_DIM = 200
N_BBLK = B_DIM // 128


def _conv_body(tbl_t, tail2, conv, in_v, out_v, sems, osems):
    wid = lax.axis_index("s") * NC + lax.axis_index("c")
    iota = lax.iota(jnp.int32, 16)
    prow = [(iota + 16 * g) >> 1 for g in range(8)]
    pcol = [((iota + 16 * g) & 1) << 6 for g in range(8)]
    colv = [iota + 16 * g for g in range(8)]

    def transpose_col(b):
        in_ref, out_ref = in_v[b], out_v[b]

        @plsc.parallel_loop(0, 16, unroll=4)
        def _(c):
            dbase = (iota + c) & 15
            for g in range(8):
                for m in range(4):
                    dvec = dbase + 16 * m
                    v = plsc.load_gather(in_ref, [dvec, colv[g]]) * SCALE
                    plsc.store_scatter(out_ref, [prow[g], pcol[g] + dvec], v)

    def fire(k, b):
        tc = wid + NW * k
        pltpu.async_copy(tbl_t.at[:, pl.ds(tc * 128, 128)], in_v[b], sems[b])

    def drain(b):
        pltpu.make_async_copy(tbl_t.at[:, pl.ds(0, 128)], in_v[b],
                              sems[b]).wait()

    def drain_store(b):
        pltpu.make_async_copy(tbl_t.at[:, pl.ds(0, 128)], out_v[b],
                              osems[b]).wait()

    n_k = (N_TCOL - 1) // NW + 1

    fire(0, 0)

    def pair_body(cp, _):
        for b in range(2):
            k = 2 * cp + b
            tc = wid + NW * k

            @pl.when(tc < N_TCOL)
            def _():
                @pl.when(k >= 2)
                def _():
                    drain_store(b)

                @pl.when(tc + NW < N_TCOL)
                def _():
                    fire(k + 1, 1 - b)

                drain(b)
                transpose_col(b)
                pltpu.async_copy(out_v[b], conv.at[pl.ds(tc * 64, 64)],
                                 osems[b])
        return ()

    lax.fori_loop(0, (n_k + 1) // 2, pair_body, ())
    drain_store(0)
    drain_store(1)

    @pl.when(wid == 0)
    def _():
        pltpu.sync_copy(tail2, in_v[0].at[pl.ds(0, 32)])

        @plsc.parallel_loop(0, 32, unroll=8)
        def _(p):
            for g in range(8):
                sl = pl.ds(16 * g, 16)
                in_v[0][p, sl] = in_v[0][p, sl] * SCALE

        pltpu.sync_copy(in_v[0].at[pl.ds(0, 32)],
                        conv.at[pl.ds(TAIL_I0 // 2, 32)])


def _gather_body(xt, conv2, o5, idx_v, rows_v, out_v, isems, gsems, osems):
    wid = lax.axis_index("s") * NC + lax.axis_index("c")
    iota = lax.iota(jnp.int32, 16)
    tvecs = [iota + 16 * g for g in range(8)]

    def fire_idx(j, b):
        pltpu.async_copy(xt.at[pl.ds(j, 1), pl.ds(wid * 128, 128)],
                         idx_v[b], isems[b])

    def drain_idx(b):
        pltpu.make_async_copy(xt.at[pl.ds(0, 1), pl.ds(0, 128)], idx_v[b],
                              isems[b]).wait()

    def fire_gather(b):
        pltpu.async_copy(conv2.at[idx_v[b].at[0]], rows_v[b], gsems[b])

    def drain_gather(b):
        pltpu.make_async_copy(conv2.at[pl.ds(0, 128)], rows_v[b],
                              gsems[b]).wait()

    def drain_store(b):
        for tr in range(8):
            pltpu.make_async_copy(o5.at[0, tr, 0], out_v[b].at[pl.ds(8 * tr, 8)],
                                  osems[b]).wait()

    def extract(b):
        rv, ov = rows_v[b], out_v[b]

        @plsc.parallel_loop(0, 16, unroll=2)
        def _(c):
            dbase = (iota + c) & 15
            for g in range(8):
                tv = tvecs[g]
                for m in range(4):
                    dvec = dbase + 16 * m
                    v = plsc.load_gather(rv, [tv, dvec])
                    plsc.store_scatter(ov, [dvec, tv], v)

    def store(j, b):
        for tr in range(8):
            pltpu.async_copy(out_v[b].at[pl.ds(8 * tr, 8)],
                             o5.at[j, tr, wid], osems[b])

    fire_idx(0, 0)
    drain_idx(0)
    fire_gather(0)
    fire_idx(1, 1)

    def pair_body(cp, _):
        for b in range(2):
            j = 2 * cp + b

            @pl.when(j + 1 < J_DIM)
            def _():
                drain_idx(1 - b)
                fire_gather(1 - b)

            @pl.when(j + 2 < J_DIM)
            def _():
                fire_idx(j + 2, b)

            @pl.when(j >= 2)
            def _():
                drain_store(b)

            drain_gather(b)
            extract(b)
            store(j, b)
        return ()

    lax.fori_loop(0, J_DIM // 2, pair_body, ())
    drain_store(0)
    drain_store(1)


def kernel(x, table):
    mesh = plsc.VectorSubcoreMesh(core_axis_name="c", subcore_axis_name="s")

    conv = pl.kernel(
        _conv_body,
        out_type=jax.ShapeDtypeStruct((CONV_ROWS, 128), jnp.float32),
        mesh=mesh,
        scratch_types=[
            [pltpu.VMEM((D_MODEL, 128), jnp.float32) for _ in range(2)],
            [pltpu.VMEM((D_MODEL, 128), jnp.float32) for _ in range(2)],
            [pltpu.SemaphoreType.DMA for _ in range(2)],
            [pltpu.SemaphoreType.DMA for _ in range(2)],
        ],
        compiler_params=pltpu.CompilerParams(use_tc_tiling_on_sc=True,
                                             needs_layout_passes=False),
    )(table.T, table[TAIL_I0:].reshape(32, 128))

    conv2 = conv.reshape(VOCAB, D_MODEL)
    xt = x.T.astype(jnp.int32)

    o5 = pl.kernel(
        _gather_body,
        out_type=jax.ShapeDtypeStruct((J_DIM, 8, N_BBLK, 8, 128), jnp.float32),
        mesh=mesh,
        scratch_types=[
            [pltpu.VMEM((1, 128), jnp.int32) for _ in range(2)],
            [pltpu.VMEM((128, D_MODEL), jnp.float32) for _ in range(2)],
            [pltpu.VMEM((D_MODEL, 128), jnp.float32) for _ in range(2)],
            [pltpu.SemaphoreType.DMA for _ in range(2)],
            [pltpu.SemaphoreType.DMA for _ in range(2)],
            [pltpu.SemaphoreType.DMA for _ in range(2)],
        ],
        compiler_params=pltpu.CompilerParams(use_tc_tiling_on_sc=False,
                                             needs_layout_passes=False),
    )(xt, conv2)

    return o5.transpose(2, 4, 0, 1, 3).reshape(B_DIM, J_DIM, D_MODEL)
